# baseline (speedup 1.0000x reference)
import jax
import jax.numpy as jnp
from jax import lax
from jax.experimental import pallas as pl
from jax.experimental.pallas import tpu as pltpu
from jax.experimental.pallas import tpu_sc as plsc

N = 50000
NP = 50048
C = 2
F = 32
E = 800000
NS = 12500
NSP = 12544
SR = NSP // 128
RA_ROWS = 7
LW = 128
G = 4
NCHUNK = 1564
ERP = NCHUNK * G
EPAD = ERP * LW - E
NT = 16
RPT_Z = NP // NT
RB = 2176
NB = NP // RB
CPT = NCHUNK // NT
CEXTRA = NCHUNK - NT * CPT


def _fin_body(af0_ref, aw0_ref, af1_ref, aw1_ref, n0_ref, n1_ref, fp_ref,
              o_ref):
    fpb = fp_ref[...] > 0.5
    outs = []
    for afr, awr, nfr in ((af0_ref, aw0_ref, n0_ref),
                          (af1_ref, aw1_ref, n1_ref)):
        interp = afr[...] / jnp.maximum(awr[...], 1e-10)
        outs.append(jnp.where(fpb, nfr[...], interp))
    o_ref[...] = jnp.concatenate(outs, axis=1)


def _sc_prep_body(sf0, sf1, comb0, comb1, ewN, z2, z1,
                  vf0, vw0, vf1, vw1, n0, n1, flag,
                  cbuf, rowbuf, ewrow, wmbuf, onesb,
                  sem_g, sem_s):
    c = lax.axis_index("c")
    s = lax.axis_index("s")

    def zfill0():
        pltpu.sync_copy(z2, n0.at[pl.ds(s * RPT_Z, RPT_Z)])
        pltpu.sync_copy(z2, vf0.at[pl.ds(s * RPT_Z, RPT_Z)])
        pltpu.sync_copy(z1, vw0.at[pl.ds(s * RPT_Z, RPT_Z)])
        pltpu.sync_copy(z1, flag.at[pl.ds(s * RPT_Z, RPT_Z)])

    def zfill1():
        pltpu.sync_copy(z2, n1.at[pl.ds(s * RPT_Z, RPT_Z)])
        pltpu.sync_copy(z2, vf1.at[pl.ds(s * RPT_Z, RPT_Z)])
        pltpu.sync_copy(z1, vw1.at[pl.ds(s * RPT_Z, RPT_Z)])

    pl.when(c == 0)(zfill0)
    pl.when(c == 1)(zfill1)
    for j in range(8):
        onesb[pl.ds(16 * j, 16)] = jnp.full((16,), 1.0, jnp.float32)
    plsc.subcore_barrier()

    def scatter_phase(sfc, combc, nc, vfc, vwc, do_flag):
        r0 = (s * (SR - RA_ROWS)) // (NT - 1)

        def row(r, carry):
            pltpu.sync_copy(combc.at[r], cbuf)
            hg = [pltpu.async_copy(sfc.at[cbuf.at[1]], rowbuf, sem_g),
                  pltpu.async_copy(ewN.at[cbuf.at[0]], ewrow, sem_g)]
            for h in hg:
                h.wait()
            hr = [pltpu.async_copy(rowbuf, nc.at[cbuf.at[0]], sem_s)]
            if do_flag:
                hr.append(pltpu.async_copy(onesb, flag.at[cbuf.at[0]],
                                           sem_s))
            for g in range(8):
                ev = ewrow[pl.ds(g * 16, 16)] + 1e-10
                mv = cbuf[2, pl.ds(g * 16, 16)].astype(jnp.float32)
                wmbuf[pl.ds(g * 16, 16)] = mv / (ev * ev)
            for h in hr:
                h.wait()

            def scale(rr, carry2):
                wm = plsc.load_gather(wmbuf, [jnp.full((16,), rr,
                                                       jnp.int32)])
                rowbuf[rr, pl.ds(0, 16)] = rowbuf[rr, pl.ds(0, 16)] * wm
                rowbuf[rr, pl.ds(16, 16)] = rowbuf[rr, pl.ds(16, 16)] * wm
                return carry2

            lax.fori_loop(0, LW, scale, 0)
            hv = [pltpu.async_copy(rowbuf, vfc.at[cbuf.at[0]], sem_s),
                  pltpu.async_copy(wmbuf, vwc.at[cbuf.at[0]], sem_s)]
            for h in hv:
                h.wait()
            return carry

        lax.fori_loop(r0, r0 + RA_ROWS, row, 0)

    pl.when(c == 0)(lambda: scatter_phase(sf0, comb0, n0, vf0, vw0, True))
    pl.when(c == 1)(lambda: scatter_phase(sf1, comb1, n1, vf1, vw1, False))


def _sc_body(vf0, vw0, vf1, vw1, srcs2, dsts2, z2, z1,
             af0, aw0, af1, aw1,
             accf, accw, sbuf, dbuf, frows, wrows, sem_g, sem_s):
    c = lax.axis_index("c")
    s = lax.axis_index("s")

    pltpu.sync_copy(z2, accf.at[pl.ds(s * RPT_Z, RPT_Z)])
    pltpu.sync_copy(z1, accw.at[pl.ds(s * RPT_Z, RPT_Z)])
    plsc.subcore_barrier()

    def phase_b(vf, vw):
        c0 = s * CPT + jnp.minimum(s, CEXTRA)
        cnt = CPT + (s < CEXTRA).astype(jnp.int32)

        def chunk(i, carry):
            base = (c0 + i) * G
            pltpu.sync_copy(srcs2.at[pl.ds(base, G)], sbuf)
            pltpu.sync_copy(dsts2.at[pl.ds(base, G)], dbuf)
            hs = [pltpu.async_copy(vf.at[sbuf.at[j]], frows.at[j], sem_g)
                  for j in range(G)]
            hs += [pltpu.async_copy(vw.at[sbuf.at[j]], wrows.at[j], sem_g)
                   for j in range(G)]
            for h in hs:
                h.wait()
            hs2 = [pltpu.async_copy(frows.at[j], accf.at[dbuf.at[j]], sem_s,
                                    add=True)
                   for j in range(G)]
            hs2 += [pltpu.async_copy(wrows.at[j], accw.at[dbuf.at[j]], sem_s,
                                     add=True)
                    for j in range(G)]
            for h in hs2:
                h.wait()
            return carry

        lax.fori_loop(0, cnt, chunk, 0)

    pl.when(c == 0)(lambda: phase_b(vf0, vw0))
    pl.when(c == 1)(lambda: phase_b(vf1, vw1))
    plsc.subcore_barrier()

    def copyout(outf, outw):
        pltpu.sync_copy(accf.at[pl.ds(s * RPT_Z, RPT_Z)],
                        outf.at[pl.ds(s * RPT_Z, RPT_Z)])
        pltpu.sync_copy(accw.at[pl.ds(s * RPT_Z, RPT_Z)],
                        outw.at[pl.ds(s * RPT_Z, RPT_Z)])

    pl.when(c == 0)(lambda: copyout(af0, aw0))
    pl.when(c == 1)(lambda: copyout(af1, aw1))


@jax.jit
def kernel(src_features, fp_idx, edge_index, edge_w):
    m0 = jnp.any(src_features[:, 0, :] != 0, axis=1).astype(jnp.float32)
    m1 = jnp.any(src_features[:, 1, :] != 0, axis=1).astype(jnp.float32)
    iota = jnp.arange(NS, dtype=jnp.int32)
    sfp, perm, mk0, mk1 = lax.sort((fp_idx, iota, m0, m1), num_keys=1,
                                   is_stable=True)
    iswin = jnp.concatenate([sfp[:-1] != sfp[1:],
                             jnp.ones((1,), bool)])
    tgt = jnp.where(iswin, sfp, N)
    tgt2 = jnp.concatenate(
        [tgt, jnp.full((NSP - NS,), N, jnp.int32)]).reshape(SR, LW)
    perm2 = jnp.concatenate(
        [perm, jnp.zeros((NSP - NS,), jnp.int32)]).reshape(SR, LW)
    mk02 = jnp.concatenate(
        [mk0.astype(jnp.int32),
         jnp.zeros((NSP - NS,), jnp.int32)]).reshape(SR, LW)
    mk12 = jnp.concatenate(
        [mk1.astype(jnp.int32),
         jnp.zeros((NSP - NS,), jnp.int32)]).reshape(SR, LW)
    comb0 = jnp.stack([tgt2, perm2, mk02], axis=1)
    comb1 = jnp.stack([tgt2, perm2, mk12], axis=1)
    sf0 = jnp.pad(src_features[:, 0, :], ((0, NSP - NS), (0, 0)))
    sf1 = jnp.pad(src_features[:, 1, :], ((0, NSP - NS), (0, 0)))
    ewN = jnp.pad(edge_w[:N, 0], (0, NP - N))
    srcs2 = jnp.concatenate(
        [edge_index[0], jnp.full((EPAD,), N, jnp.int32)]).reshape(ERP, LW)
    dsts2 = jnp.concatenate(
        [edge_index[1], jnp.full((EPAD,), N, jnp.int32)]).reshape(ERP, LW)
    z2 = jnp.zeros((RPT_Z, F), jnp.float32)
    z1 = jnp.zeros((RPT_Z,), jnp.float32)

    mesh = plsc.VectorSubcoreMesh(core_axis_name="c", subcore_axis_name="s",
                                  num_cores=2, num_subcores=NT)

    vf0, vw0, vf1, vw1, n0, n1, flag = pl.kernel(
        _sc_prep_body,
        out_type=[jax.ShapeDtypeStruct((NP, F), jnp.float32),
                  jax.ShapeDtypeStruct((NP,), jnp.float32)] * 2 +
                 [jax.ShapeDtypeStruct((NP, F), jnp.float32)] * 2 +
                 [jax.ShapeDtypeStruct((NP,), jnp.float32)],
        mesh=mesh,
        compiler_params=pltpu.CompilerParams(use_tc_tiling_on_sc=False,
                                             needs_layout_passes=False),
        scratch_types=[
            pltpu.VMEM((3, LW), jnp.int32),
            pltpu.VMEM((LW, F), jnp.float32),
            pltpu.VMEM((LW,), jnp.float32),
            pltpu.VMEM((LW,), jnp.float32),
            pltpu.VMEM((LW,), jnp.float32),
            pltpu.SemaphoreType.DMA,
            pltpu.SemaphoreType.DMA,
        ],
    )(sf0, sf1, comb0, comb1, ewN, z2, z1)

    af0, aw0, af1, aw1 = pl.kernel(
        _sc_body,
        out_type=[jax.ShapeDtypeStruct((NP, F), jnp.float32),
                  jax.ShapeDtypeStruct((NP,), jnp.float32)] * 2,
        mesh=mesh,
        compiler_params=pltpu.CompilerParams(use_tc_tiling_on_sc=False),
        scratch_types=[
            pltpu.VMEM_SHARED((NP, F), jnp.float32),
            pltpu.VMEM_SHARED((NP,), jnp.float32),
            pltpu.VMEM((G, LW), jnp.int32),
            pltpu.VMEM((G, LW), jnp.int32),
            pltpu.VMEM((G, LW, F), jnp.float32),
            pltpu.VMEM((G, LW), jnp.float32),
            pltpu.SemaphoreType.DMA,
            pltpu.SemaphoreType.DMA,
        ],
    )(vf0, vw0, vf1, vw1, srcs2, dsts2, z2, z1)

    outflat = pl.pallas_call(
        _fin_body,
        grid=(NB,),
        in_specs=[pl.BlockSpec((RB, F), lambda i: (i, 0)),
                  pl.BlockSpec((RB, 1), lambda i: (i, 0)),
                  pl.BlockSpec((RB, F), lambda i: (i, 0)),
                  pl.BlockSpec((RB, 1), lambda i: (i, 0)),
                  pl.BlockSpec((RB, F), lambda i: (i, 0)),
                  pl.BlockSpec((RB, F), lambda i: (i, 0)),
                  pl.BlockSpec((RB, 1), lambda i: (i, 0))],
        out_specs=pl.BlockSpec((RB, C * F), lambda i: (i, 0)),
        out_shape=jax.ShapeDtypeStruct((NP, C * F), jnp.float32),
    )(af0, aw0.reshape(NP, 1), af1, aw1.reshape(NP, 1), n0, n1,
      flag.reshape(NP, 1))
    return outflat[:N].reshape(N, C, F)


# speedup vs baseline: 1.2108x; 1.2108x over previous
import jax
import jax.numpy as jnp
from jax import lax
from jax.experimental import pallas as pl
from jax.experimental.pallas import tpu as pltpu
from jax.experimental.pallas import tpu_sc as plsc

N = 50000
NP = 50048
C = 2
F = 32
E = 800000
NS = 12500
NSP = 12544
SR = NSP // 128
RA_ROWS = 7
LW = 128
G = 4
NCHUNK = 1564
ERP = NCHUNK * G
EPAD = ERP * LW - E
NT = 16
RPT_Z = NP // NT
RB = 2176
NB = NP // RB
CPT = NCHUNK // NT
CEXTRA = NCHUNK - NT * CPT


def _fin_body(af0_ref, aw0_ref, af1_ref, aw1_ref, n0_ref, n1_ref, fp_ref,
              o_ref):
    fpb = fp_ref[...] > 0.5
    outs = []
    for afr, awr, nfr in ((af0_ref, aw0_ref, n0_ref),
                          (af1_ref, aw1_ref, n1_ref)):
        interp = afr[...] / jnp.maximum(awr[...], 1e-10)
        outs.append(jnp.where(fpb, nfr[...], interp))
    o_ref[...] = jnp.concatenate(outs, axis=1)


def _prep_body(n0_ref, n1_ref, ew_ref, f0_ref, w0_ref, f1_ref, w1_ref):
    ew = ew_ref[...] + 1e-10
    w = 1.0 / (ew * ew)
    for nref, fref, wref in ((n0_ref, f0_ref, w0_ref),
                             (n1_ref, f1_ref, w1_ref)):
        f = nref[...]
        m = jnp.any(f != 0, axis=1, keepdims=True)
        wm = jnp.where(m, w, 0.0)
        fref[...] = f * wm
        wref[...] = wm


def _sc_scatter_body(sf0, sf1, comb0, comb1, ones1, z2, z1,
                     n0, n1, flag,
                     cbuf, rowbuf, onesb, sem_g, sem_s):
    c = lax.axis_index("c")
    s = lax.axis_index("s")

    def zfill0():
        pltpu.sync_copy(z2, n0.at[pl.ds(s * RPT_Z, RPT_Z)])
        pltpu.sync_copy(z1, flag.at[pl.ds(s * RPT_Z, RPT_Z)])

    def zfill1():
        pltpu.sync_copy(z2, n1.at[pl.ds(s * RPT_Z, RPT_Z)])

    pl.when(c == 0)(zfill0)
    pl.when(c == 1)(zfill1)
    pltpu.sync_copy(ones1, onesb)
    plsc.subcore_barrier()

    def scatter_phase(sfc, combc, nc, do_flag):
        r0 = (s * (SR - RA_ROWS)) // (NT - 1)

        def row(r, carry):
            pltpu.sync_copy(combc.at[r], cbuf)
            pltpu.async_copy(sfc.at[cbuf.at[1]], rowbuf, sem_g).wait()
            hr = [pltpu.async_copy(rowbuf, nc.at[cbuf.at[0]], sem_s)]
            if do_flag:
                hr.append(pltpu.async_copy(onesb.at[0],
                                           flag.at[cbuf.at[0]], sem_s))
            for h in hr:
                h.wait()
            return carry

        lax.fori_loop(r0, r0 + RA_ROWS, row, 0)

    pl.when(c == 0)(lambda: scatter_phase(sf0, comb0, n0, True))
    pl.when(c == 1)(lambda: scatter_phase(sf1, comb1, n1, False))


def _sc_body(vf0, vw0, vf1, vw1, srcs2, dsts2, z2, z1,
             af0, aw0, af1, aw1,
             accf, accw, sbuf, dbuf, frows, wrows, sem_g, sem_s):
    c = lax.axis_index("c")
    s = lax.axis_index("s")

    pltpu.sync_copy(z2, accf.at[pl.ds(s * RPT_Z, RPT_Z)])
    pltpu.sync_copy(z1, accw.at[pl.ds(s * RPT_Z, RPT_Z)])
    plsc.subcore_barrier()

    def phase_b(vf, vw):
        c0 = s * CPT + jnp.minimum(s, CEXTRA)
        cnt = CPT + (s < CEXTRA).astype(jnp.int32)

        def chunk(i, carry):
            base = (c0 + i) * G
            pltpu.sync_copy(srcs2.at[pl.ds(base, G)], sbuf)
            pltpu.sync_copy(dsts2.at[pl.ds(base, G)], dbuf)
            hs = [pltpu.async_copy(vf.at[sbuf.at[j]], frows.at[j], sem_g)
                  for j in range(G)]
            hs += [pltpu.async_copy(vw.at[sbuf.at[j]], wrows.at[j], sem_g)
                   for j in range(G)]
            for h in hs:
                h.wait()
            hs2 = [pltpu.async_copy(frows.at[j], accf.at[dbuf.at[j]], sem_s,
                                    add=True)
                   for j in range(G)]
            hs2 += [pltpu.async_copy(wrows.at[j], accw.at[dbuf.at[j]], sem_s,
                                     add=True)
                    for j in range(G)]
            for h in hs2:
                h.wait()
            return carry

        lax.fori_loop(0, cnt, chunk, 0)

    pl.when(c == 0)(lambda: phase_b(vf0, vw0))
    pl.when(c == 1)(lambda: phase_b(vf1, vw1))
    plsc.subcore_barrier()

    def copyout(outf, outw):
        pltpu.sync_copy(accf.at[pl.ds(s * RPT_Z, RPT_Z)],
                        outf.at[pl.ds(s * RPT_Z, RPT_Z)])
        pltpu.sync_copy(accw.at[pl.ds(s * RPT_Z, RPT_Z)],
                        outw.at[pl.ds(s * RPT_Z, RPT_Z)])

    pl.when(c == 0)(lambda: copyout(af0, aw0))
    pl.when(c == 1)(lambda: copyout(af1, aw1))


@jax.jit
def kernel(src_features, fp_idx, edge_index, edge_w):
    m0 = jnp.any(src_features[:, 0, :] != 0, axis=1).astype(jnp.float32)
    m1 = jnp.any(src_features[:, 1, :] != 0, axis=1).astype(jnp.float32)
    iota = jnp.arange(NS, dtype=jnp.int32)
    sfp, perm, mk0, mk1 = lax.sort((fp_idx, iota, m0, m1), num_keys=1,
                                   is_stable=True)
    iswin = jnp.concatenate([sfp[:-1] != sfp[1:],
                             jnp.ones((1,), bool)])
    tgt = jnp.where(iswin, sfp, N)
    tgt2 = jnp.concatenate(
        [tgt, jnp.full((NSP - NS,), N, jnp.int32)]).reshape(SR, LW)
    perm2 = jnp.concatenate(
        [perm, jnp.zeros((NSP - NS,), jnp.int32)]).reshape(SR, LW)
    mk02 = jnp.concatenate(
        [mk0.astype(jnp.int32),
         jnp.zeros((NSP - NS,), jnp.int32)]).reshape(SR, LW)
    mk12 = jnp.concatenate(
        [mk1.astype(jnp.int32),
         jnp.zeros((NSP - NS,), jnp.int32)]).reshape(SR, LW)
    comb0 = jnp.stack([tgt2, perm2, mk02], axis=1)
    comb1 = jnp.stack([tgt2, perm2, mk12], axis=1)
    sf0 = jnp.pad(src_features[:, 0, :], ((0, NSP - NS), (0, 0)))
    sf1 = jnp.pad(src_features[:, 1, :], ((0, NSP - NS), (0, 0)))
    ewN = jnp.pad(edge_w[:N, 0], (0, NP - N))
    srcs2 = jnp.concatenate(
        [edge_index[0], jnp.full((EPAD,), N, jnp.int32)]).reshape(ERP, LW)
    dsts2 = jnp.concatenate(
        [edge_index[1], jnp.full((EPAD,), N, jnp.int32)]).reshape(ERP, LW)
    z2 = jnp.zeros((RPT_Z, F), jnp.float32)
    z1 = jnp.zeros((RPT_Z,), jnp.float32)
    ones1 = jnp.ones((1, LW), jnp.float32)

    mesh = plsc.VectorSubcoreMesh(core_axis_name="c", subcore_axis_name="s",
                                  num_cores=2, num_subcores=NT)

    n0, n1, flag = pl.kernel(
        _sc_scatter_body,
        out_type=[jax.ShapeDtypeStruct((NP, F), jnp.float32)] * 2 +
                 [jax.ShapeDtypeStruct((NP,), jnp.float32)],
        mesh=mesh,
        compiler_params=pltpu.CompilerParams(use_tc_tiling_on_sc=False),
        scratch_types=[
            pltpu.VMEM((3, LW), jnp.int32),
            pltpu.VMEM((LW, F), jnp.float32),
            pltpu.VMEM((1, LW), jnp.float32),
            pltpu.SemaphoreType.DMA,
            pltpu.SemaphoreType.DMA,
        ],
    )(sf0, sf1, comb0, comb1, ones1, z2, z1)

    vf0, vw0, vf1, vw1 = pl.pallas_call(
        _prep_body,
        grid=(NB,),
        in_specs=[pl.BlockSpec((RB, F), lambda i: (i, 0)),
                  pl.BlockSpec((RB, F), lambda i: (i, 0)),
                  pl.BlockSpec((RB, 1), lambda i: (i, 0))],
        out_specs=[pl.BlockSpec((RB, F), lambda i: (i, 0)),
                   pl.BlockSpec((RB, 1), lambda i: (i, 0))] * 2,
        out_shape=[jax.ShapeDtypeStruct((NP, F), jnp.float32),
                   jax.ShapeDtypeStruct((NP, 1), jnp.float32)] * 2,
    )(n0, n1, ewN.reshape(NP, 1))
    vw0 = vw0.reshape(NP)
    vw1 = vw1.reshape(NP)

    af0, aw0, af1, aw1 = pl.kernel(
        _sc_body,
        out_type=[jax.ShapeDtypeStruct((NP, F), jnp.float32),
                  jax.ShapeDtypeStruct((NP,), jnp.float32)] * 2,
        mesh=mesh,
        compiler_params=pltpu.CompilerParams(use_tc_tiling_on_sc=False),
        scratch_types=[
            pltpu.VMEM_SHARED((NP, F), jnp.float32),
            pltpu.VMEM_SHARED((NP,), jnp.float32),
            pltpu.VMEM((G, LW), jnp.int32),
            pltpu.VMEM((G, LW), jnp.int32),
            pltpu.VMEM((G, LW, F), jnp.float32),
            pltpu.VMEM((G, LW), jnp.float32),
            pltpu.SemaphoreType.DMA,
            pltpu.SemaphoreType.DMA,
        ],
    )(vf0, vw0, vf1, vw1, srcs2, dsts2, z2, z1)

    outflat = pl.pallas_call(
        _fin_body,
        grid=(NB,),
        in_specs=[pl.BlockSpec((RB, F), lambda i: (i, 0)),
                  pl.BlockSpec((RB, 1), lambda i: (i, 0)),
                  pl.BlockSpec((RB, F), lambda i: (i, 0)),
                  pl.BlockSpec((RB, 1), lambda i: (i, 0)),
                  pl.BlockSpec((RB, F), lambda i: (i, 0)),
                  pl.BlockSpec((RB, F), lambda i: (i, 0)),
                  pl.BlockSpec((RB, 1), lambda i: (i, 0))],
        out_specs=pl.BlockSpec((RB, C * F), lambda i: (i, 0)),
        out_shape=jax.ShapeDtypeStruct((NP, C * F), jnp.float32),
    )(af0, aw0.reshape(NP, 1), af1, aw1.reshape(NP, 1), n0, n1,
      flag.reshape(NP, 1))
    return outflat[:N].reshape(N, C, F)


# speedup vs baseline: 1.4638x; 1.2090x over previous
import jax
import jax.numpy as jnp
from jax import lax
from jax.experimental import pallas as pl
from jax.experimental.pallas import tpu as pltpu
from jax.experimental.pallas import tpu_sc as plsc

N = 50000
NP = 50048
C = 2
F = 32
E = 800000
NS = 12500
NSP = 12544
SR = NSP // 128
RA_ROWS = 7
LW = 128
G = 4
NCHUNK = 1564
ERP = NCHUNK * G
EPAD = ERP * LW - E
NT = 16
RPT_Z = NP // NT
RB = 2176
NB = NP // RB
CPT = NCHUNK // NT
CEXTRA = NCHUNK - NT * CPT


def _fin_body(af0_ref, aw0_ref, af1_ref, aw1_ref, n0_ref, n1_ref, fp_ref,
              o_ref):
    fpb = fp_ref[...] > 0.5
    outs = []
    for afr, awr, nfr in ((af0_ref, aw0_ref, n0_ref),
                          (af1_ref, aw1_ref, n1_ref)):
        interp = afr[...] / jnp.maximum(awr[...], 1e-10)
        outs.append(jnp.where(fpb, nfr[...], interp))
    o_ref[...] = jnp.concatenate(outs, axis=1)


def _prep_body(n0_ref, n1_ref, ew_ref, f0_ref, w0_ref, f1_ref, w1_ref,
               fl_ref):
    ew = ew_ref[...] + 1e-10
    w = 1.0 / (ew * ew)
    ms = []
    for nref, fref, wref in ((n0_ref, f0_ref, w0_ref),
                             (n1_ref, f1_ref, w1_ref)):
        f = nref[...]
        m = jnp.any(f != 0, axis=1, keepdims=True)
        ms.append(m)
        wm = jnp.where(m, w, 0.0)
        fref[...] = f * wm
        wref[...] = wm
    fl_ref[...] = (ms[0] | ms[1]).astype(jnp.float32)


def _sc_scatter_body(sf0, sf1, comb0, comb1, z2,
                     n0, n1,
                     cbuf, rowbuf, sem_g, sem_s):
    c = lax.axis_index("c")
    s = lax.axis_index("s")

    pl.when(c == 0)(lambda: pltpu.sync_copy(
        z2, n0.at[pl.ds(s * RPT_Z, RPT_Z)]))
    pl.when(c == 1)(lambda: pltpu.sync_copy(
        z2, n1.at[pl.ds(s * RPT_Z, RPT_Z)]))
    plsc.subcore_barrier()

    def scatter_phase(sfc, combc, nc):
        r0 = (s * (SR - RA_ROWS)) // (NT - 1)

        def row(r, carry):
            pltpu.sync_copy(combc.at[r], cbuf)
            pltpu.async_copy(sfc.at[cbuf.at[1]], rowbuf, sem_g).wait()
            pltpu.async_copy(rowbuf, nc.at[cbuf.at[0]], sem_s).wait()
            return carry

        lax.fori_loop(r0, r0 + RA_ROWS, row, 0)

    pl.when(c == 0)(lambda: scatter_phase(sf0, comb0, n0))
    pl.when(c == 1)(lambda: scatter_phase(sf1, comb1, n1))


def _sc_body(vf0, vw0, vf1, vw1, srcs2, dsts2, z2, z1,
             af0, aw0, af1, aw1,
             accf, accw, sbuf, dbuf, frows, wrows, sem_g, sem_s):
    c = lax.axis_index("c")
    s = lax.axis_index("s")

    pltpu.sync_copy(z2, accf.at[pl.ds(s * RPT_Z, RPT_Z)])
    pltpu.sync_copy(z1, accw.at[pl.ds(s * RPT_Z, RPT_Z)])
    plsc.subcore_barrier()

    def phase_b(vf, vw):
        c0 = s * CPT + jnp.minimum(s, CEXTRA)
        cnt = CPT + (s < CEXTRA).astype(jnp.int32)

        def chunk(i, carry):
            base = (c0 + i) * G
            pltpu.sync_copy(srcs2.at[pl.ds(base, G)], sbuf)
            pltpu.sync_copy(dsts2.at[pl.ds(base, G)], dbuf)
            hs = [pltpu.async_copy(vf.at[sbuf.at[j]], frows.at[j], sem_g)
                  for j in range(G)]
            hs += [pltpu.async_copy(vw.at[sbuf.at[j]], wrows.at[j], sem_g)
                   for j in range(G)]
            for h in hs:
                h.wait()
            hs2 = [pltpu.async_copy(frows.at[j], accf.at[dbuf.at[j]], sem_s,
                                    add=True)
                   for j in range(G)]
            hs2 += [pltpu.async_copy(wrows.at[j], accw.at[dbuf.at[j]], sem_s,
                                     add=True)
                    for j in range(G)]
            for h in hs2:
                h.wait()
            return carry

        lax.fori_loop(0, cnt, chunk, 0)

    pl.when(c == 0)(lambda: phase_b(vf0, vw0))
    pl.when(c == 1)(lambda: phase_b(vf1, vw1))
    plsc.subcore_barrier()

    def copyout(outf, outw):
        pltpu.sync_copy(accf.at[pl.ds(s * RPT_Z, RPT_Z)],
                        outf.at[pl.ds(s * RPT_Z, RPT_Z)])
        pltpu.sync_copy(accw.at[pl.ds(s * RPT_Z, RPT_Z)],
                        outw.at[pl.ds(s * RPT_Z, RPT_Z)])

    pl.when(c == 0)(lambda: copyout(af0, aw0))
    pl.when(c == 1)(lambda: copyout(af1, aw1))


@jax.jit
def kernel(src_features, fp_idx, edge_index, edge_w):
    m0 = jnp.any(src_features[:, 0, :] != 0, axis=1).astype(jnp.float32)
    m1 = jnp.any(src_features[:, 1, :] != 0, axis=1).astype(jnp.float32)
    iota = jnp.arange(NS, dtype=jnp.int32)
    sfp, perm, mk0, mk1 = lax.sort((fp_idx, iota, m0, m1), num_keys=1,
                                   is_stable=True)
    iswin = jnp.concatenate([sfp[:-1] != sfp[1:],
                             jnp.ones((1,), bool)])
    tgt = jnp.where(iswin, sfp, N)
    tgt2 = jnp.concatenate(
        [tgt, jnp.full((NSP - NS,), N, jnp.int32)]).reshape(SR, LW)
    perm2 = jnp.concatenate(
        [perm, jnp.zeros((NSP - NS,), jnp.int32)]).reshape(SR, LW)
    mk02 = jnp.concatenate(
        [mk0.astype(jnp.int32),
         jnp.zeros((NSP - NS,), jnp.int32)]).reshape(SR, LW)
    mk12 = jnp.concatenate(
        [mk1.astype(jnp.int32),
         jnp.zeros((NSP - NS,), jnp.int32)]).reshape(SR, LW)
    comb0 = jnp.stack([tgt2, perm2, mk02], axis=1)
    comb1 = jnp.stack([tgt2, perm2, mk12], axis=1)
    sf0 = jnp.pad(src_features[:, 0, :], ((0, NSP - NS), (0, 0)))
    sf1 = jnp.pad(src_features[:, 1, :], ((0, NSP - NS), (0, 0)))
    ewN = jnp.pad(edge_w[:N, 0], (0, NP - N))
    srcs2 = jnp.concatenate(
        [edge_index[0], jnp.full((EPAD,), N, jnp.int32)]).reshape(ERP, LW)
    dsts2 = jnp.concatenate(
        [edge_index[1], jnp.full((EPAD,), N, jnp.int32)]).reshape(ERP, LW)
    z2 = jnp.zeros((RPT_Z, F), jnp.float32)
    z1 = jnp.zeros((RPT_Z,), jnp.float32)

    mesh = plsc.VectorSubcoreMesh(core_axis_name="c", subcore_axis_name="s",
                                  num_cores=2, num_subcores=NT)

    n0, n1 = pl.kernel(
        _sc_scatter_body,
        out_type=[jax.ShapeDtypeStruct((NP, F), jnp.float32)] * 2,
        mesh=mesh,
        compiler_params=pltpu.CompilerParams(use_tc_tiling_on_sc=False),
        scratch_types=[
            pltpu.VMEM((3, LW), jnp.int32),
            pltpu.VMEM((LW, F), jnp.float32),
            pltpu.SemaphoreType.DMA,
            pltpu.SemaphoreType.DMA,
        ],
    )(sf0, sf1, comb0, comb1, z2)

    vf0, vw0, vf1, vw1, flag = pl.pallas_call(
        _prep_body,
        grid=(NB,),
        in_specs=[pl.BlockSpec((RB, F), lambda i: (i, 0)),
                  pl.BlockSpec((RB, F), lambda i: (i, 0)),
                  pl.BlockSpec((RB, 1), lambda i: (i, 0))],
        out_specs=[pl.BlockSpec((RB, F), lambda i: (i, 0)),
                   pl.BlockSpec((RB, 1), lambda i: (i, 0))] * 2 +
                  [pl.BlockSpec((RB, 1), lambda i: (i, 0))],
        out_shape=[jax.ShapeDtypeStruct((NP, F), jnp.float32),
                   jax.ShapeDtypeStruct((NP, 1), jnp.float32)] * 2 +
                  [jax.ShapeDtypeStruct((NP, 1), jnp.float32)],
    )(n0, n1, ewN.reshape(NP, 1))
    vw0 = vw0.reshape(NP)
    vw1 = vw1.reshape(NP)

    af0, aw0, af1, aw1 = pl.kernel(
        _sc_body,
        out_type=[jax.ShapeDtypeStruct((NP, F), jnp.float32),
                  jax.ShapeDtypeStruct((NP,), jnp.float32)] * 2,
        mesh=mesh,
        compiler_params=pltpu.CompilerParams(use_tc_tiling_on_sc=False),
        scratch_types=[
            pltpu.VMEM_SHARED((NP, F), jnp.float32),
            pltpu.VMEM_SHARED((NP,), jnp.float32),
            pltpu.VMEM((G, LW), jnp.int32),
            pltpu.VMEM((G, LW), jnp.int32),
            pltpu.VMEM((G, LW, F), jnp.float32),
            pltpu.VMEM((G, LW), jnp.float32),
            pltpu.SemaphoreType.DMA,
            pltpu.SemaphoreType.DMA,
        ],
    )(vf0, vw0, vf1, vw1, srcs2, dsts2, z2, z1)

    outflat = pl.pallas_call(
        _fin_body,
        grid=(NB,),
        in_specs=[pl.BlockSpec((RB, F), lambda i: (i, 0)),
                  pl.BlockSpec((RB, 1), lambda i: (i, 0)),
                  pl.BlockSpec((RB, F), lambda i: (i, 0)),
                  pl.BlockSpec((RB, 1), lambda i: (i, 0)),
                  pl.BlockSpec((RB, F), lambda i: (i, 0)),
                  pl.BlockSpec((RB, F), lambda i: (i, 0)),
                  pl.BlockSpec((RB, 1), lambda i: (i, 0))],
        out_specs=pl.BlockSpec((RB, C * F), lambda i: (i, 0)),
        out_shape=jax.ShapeDtypeStruct((NP, C * F), jnp.float32),
    )(af0, aw0.reshape(NP, 1), af1, aw1.reshape(NP, 1), n0, n1, flag)
    return outflat[:N].reshape(N, C, F)
